# baseline (speedup 1.0000x reference)
import functools

import jax
import jax.numpy as jnp
from jax import lax
from jax.experimental import pallas as pl
from jax.experimental.pallas import tpu as pltpu
from jax.experimental.pallas import tpu_sc as plsc

N = 50000
E = 800000
F = 4
C = 64
H = 4

NC, NS, L = 2, 16, 16
NW = NC * NS
CH = 256
GW = 24
CHG = 256
EPT = 25088
E_PAD = EPT * NW
NCHUNK = EPT // CH
TR = 50176
NP = 51200
RPT = NP // NS
ZB = 128
F32 = jnp.float32

_mesh = plsc.VectorSubcoreMesh(core_axis_name="c", subcore_axis_name="s")
_params = pltpu.CompilerParams(use_tc_tiling_on_sc=False)


def _zero_acc(zb, acc, s, width):
    z = jnp.zeros((L,), F32)
    offs = sorted({0, width - L})

    def zrow(i, _):
        for j in offs:
            zb[i, pl.ds(j, L)] = z
        return 0
    lax.fori_loop(0, ZB, zrow, 0)

    def zcp(j, _):
        pltpu.sync_copy(zb, acc.at[pl.ds(s * RPT + j * ZB, ZB)])
        return 0
    lax.fori_loop(0, RPT // ZB, zcp, 0)


def _dump_acc(zb, acc, out, c, s):
    def dcp(j, _):
        r = s * RPT + j * ZB
        pltpu.sync_copy(acc.at[pl.ds(r, ZB)], zb)
        pltpu.sync_copy(zb, out.at[c, pl.ds(r, ZB)])
        return 0
    lax.fori_loop(0, RPT // ZB, dcp, 0)


@functools.partial(
    pl.kernel,
    out_type=jax.ShapeDtypeStruct((NC, NP, GW), F32),
    mesh=_mesh,
    compiler_params=_params,
    scratch_types=[
        pltpu.VMEM((4, CH), jnp.int32),
        pltpu.VMEM((4, CH), jnp.int32),
        pltpu.VMEM((2, CH, L), F32),
        pltpu.VMEM((2, CH, L), F32),
        pltpu.VMEM((2, CH, GW), F32),
        pltpu.VMEM((ZB, GW), F32),
        pltpu.VMEM_SHARED((NP, GW), F32),
        pltpu.SemaphoreType.DMA,
        pltpu.SemaphoreType.DMA,
        pltpu.SemaphoreType.DMA,
    ],
)
def _gat_pass(asx, ad, srcs, dsts, out,
              isx, isd, rows_s, rows_d, vals, zb, acc, isem, gsem, ssem):
    c = lax.axis_index("c")
    s = lax.axis_index("s")
    w = c * NS + s
    base = w * EPT
    _zero_acc(zb, acc, s, GW)
    plsc.subcore_barrier()

    lane = lax.iota(jnp.int32, L)
    m4 = lane < 4
    cnt = jnp.where(lane == 4, 1.0, 0.0).astype(F32)
    i1 = lax.shift_right_logical(lane, 2)
    i2 = 4 + (lane & 3)

    def idx_start(j):
        r = j & 3
        eb = base + j * CH
        pltpu.async_copy(srcs.at[pl.ds(eb, CH)], isx.at[r], isem)
        pltpu.async_copy(dsts.at[pl.ds(eb, CH)], isd.at[r], isem)

    def idx_drain(r):
        pltpu.make_async_copy(srcs.at[pl.ds(0, CH)], isx.at[r], isem).wait()
        pltpu.make_async_copy(dsts.at[pl.ds(0, CH)], isd.at[r], isem).wait()

    def gather_start(p, r):
        pltpu.async_copy(asx.at[isx.at[r]], rows_s.at[p], gsem)
        pltpu.async_copy(ad.at[isd.at[r]], rows_d.at[p], gsem)

    idx_start(0)
    idx_drain(0)
    gather_start(0, 0)
    idx_start(1)

    def body(i, _):
        p = i & 1
        q = 1 - p
        pltpu.make_async_copy(asx.at[pl.ds(0, CH)], rows_s.at[p], gsem).wait()
        pltpu.make_async_copy(ad.at[pl.ds(0, CH)], rows_d.at[p], gsem).wait()
        @pl.when(i >= 2)
        def _():
            pltpu.make_async_copy(out.at[0, pl.ds(0, CH)], vals.at[p], ssem).wait()
        @pl.when(i + 2 < NCHUNK)
        def _():
            idx_start(i + 2)
        @pl.when(i + 1 < NCHUNK)
        def _():
            idx_drain((i + 1) & 3)
            gather_start(q, (i + 1) & 3)

        def edge(e, _):
            af = rows_s[p, e, :] + rows_d[p, e, :]
            ex = jnp.exp(jnp.maximum(af, 0.2 * af))
            vals[p, e, pl.ds(0, L)] = jnp.where(m4, ex, cnt)
            g1 = ex.at[i1].get(mode="promise_in_bounds")
            g2 = af.at[i2].get(mode="promise_in_bounds")
            vals[p, e, pl.ds(GW - L, L)] = g1 * g2
            return 0
        lax.fori_loop(0, CH, edge, 0, unroll=8)
        pltpu.async_copy(vals.at[p], acc.at[isd.at[i & 3]], ssem, add=True)
        return 0
    lax.fori_loop(0, NCHUNK, body, 0)
    pltpu.make_async_copy(out.at[0, pl.ds(0, CH)], vals.at[0], ssem).wait()
    pltpu.make_async_copy(out.at[0, pl.ds(0, CH)], vals.at[1], ssem).wait()
    plsc.subcore_barrier()
    _dump_acc(zb, acc, out, c, s)


NCH_G = (E_PAD // NS) // CHG


@functools.partial(
    pl.kernel,
    out_type=jax.ShapeDtypeStruct((NC, NP, 32), F32),
    mesh=_mesh,
    compiler_params=_params,
    scratch_types=[
        pltpu.VMEM((4, CHG), jnp.int32),
        pltpu.VMEM((4, CHG), jnp.int32),
        pltpu.VMEM((2, CHG, 32), F32),
        pltpu.VMEM((ZB, 32), F32),
        pltpu.VMEM_SHARED((NP, 32), F32),
        pltpu.SemaphoreType.DMA,
        pltpu.SemaphoreType.DMA,
        pltpu.SemaphoreType.DMA,
    ],
)
def _gcn_pass(hs2, srcs2, dsts, out, isx, isd, rows, zb, acc, isem, gsem, ssem):
    c = lax.axis_index("c")
    s = lax.axis_index("s")
    base = s * (E_PAD // NS)
    _zero_acc(zb, acc, s, 32)
    plsc.subcore_barrier()

    def idx_start(j):
        r = j & 3
        eb = base + j * CHG
        pltpu.async_copy(srcs2.at[c, pl.ds(eb, CHG)], isx.at[r], isem)
        pltpu.async_copy(dsts.at[pl.ds(eb, CHG)], isd.at[r], isem)

    def idx_drain(r):
        pltpu.make_async_copy(dsts.at[pl.ds(0, CHG)], isx.at[r], isem).wait()
        pltpu.make_async_copy(dsts.at[pl.ds(0, CHG)], isd.at[r], isem).wait()

    idx_start(0)
    idx_drain(0)
    pltpu.async_copy(hs2.at[isx.at[0]], rows.at[0], gsem)
    idx_start(1)

    def body(i, _):
        p = i & 1
        pltpu.make_async_copy(hs2.at[pl.ds(0, CHG)], rows.at[p], gsem).wait()
        @pl.when(i >= 1)
        def _():
            pltpu.make_async_copy(out.at[0, pl.ds(0, CHG)], rows.at[1 - p], ssem).wait()
        @pl.when(i + 2 < NCH_G)
        def _():
            idx_start(i + 2)
        @pl.when(i + 1 < NCH_G)
        def _():
            idx_drain((i + 1) & 3)
            pltpu.async_copy(hs2.at[isx.at[(i + 1) & 3]], rows.at[1 - p], gsem)
        pltpu.async_copy(rows.at[p], acc.at[isd.at[i & 3]], ssem, add=True)
        return 0
    lax.fori_loop(0, NCH_G, body, 0)
    pltpu.make_async_copy(out.at[0, pl.ds(0, CHG)], rows.at[0], ssem).wait()
    plsc.subcore_barrier()
    _dump_acc(zb, acc, out, c, s)

BK = 3584
GRID = TR // BK


def _sel(shape, fill):
    import numpy as np
    m = np.zeros(shape, np.float32)
    fill(m)
    return m


_PD = _sel((GW, 256), lambda m: [m.__setitem__((hh, slice(hh * 64, hh * 64 + 64)), 1.0) for hh in range(4)])
_PR = _sel((4, 256), lambda m: [m.__setitem__((hh, slice(hh * 64, hh * 64 + 64)), 1.0) for hh in range(4)])
_PS = _sel((GW, 16), lambda m: [m.__setitem__((GW - 16 + k, k), 1.0) for k in range(16)])
_EH = _sel((4, 16), lambda m: [m.__setitem__((hh, hh * 4 + ff), 1.0) for hh in range(4) for ff in range(4)])
_EF = _sel((4, 16), lambda m: [m.__setitem__((ff, hh * 4 + ff), 1.0) for hh in range(4) for ff in range(4)])
_PI = _sel((GW, 8), lambda m: m.__setitem__((4, slice(None)), 1.0))
_E0 = _sel((32, 64), lambda m: [m.__setitem__((j, j), 1.0) for j in range(32)])
_E1 = _sel((32, 64), lambda m: [m.__setitem__((j, 32 + j), 1.0) for j in range(32)])
_P4 = _sel((16, 4), lambda m: [m.__setitem__((k, k), 1.0) for k in range(4)])
_P48 = _sel((16, 4), lambda m: [m.__setitem__((4 + k, k), 1.0) for k in range(4)])
_SA = _sel((4, 16), lambda m: [m.__setitem__((k, k), 1.0) for k in range(4)])
_SX = _sel((4, 16), lambda m: [m.__setitem__((k, 4 + k), 1.0) for k in range(4)])
_PRT = _PR.T.copy()
_RHT = [_sel((4, 16), lambda m, h=h: [m.__setitem__((ff, h * 4 + ff), 1.0) for ff in range(4)]) for h in range(4)]
_CHT = [_sel((256, 64), lambda m, h=h: [m.__setitem__((h * 64 + cc, cc), 1.0) for cc in range(64)]) for h in range(4)]


def _full(shape):
    return pl.BlockSpec(shape, lambda i: tuple(0 for _ in shape))


def _rows(width):
    return pl.BlockSpec((BK, width), lambda i: (i, 0))


def _core(cc, width=32):
    return pl.BlockSpec((1, BK, width), lambda i, c=cc: (c, i, 0))


def _gelu(v):
    return 0.5 * v * (1.0 + lax.erf(v * (2.0 ** -0.5)))


def _dotT(a, bT):
    return lax.dot_general(a, bT, (((1,), (1,)), ((), ())))


def _k0_body(gw, afs, afd, prt, sa, sx, rh0, rh1, rh2, rh3,
             ch0, ch1, ch2, ch3, om1, om2, owbdT):
    gwv = gw[...]
    blk_s = afs[...] * prt[...]
    blk_d = afd[...] * prt[...]
    asT = lax.dot_general(gwv, blk_s, (((0,), (0,)), ((), ())))
    adT = lax.dot_general(gwv, blk_d, (((0,), (0,)), ((), ())))
    om1[...] = asT @ sa[...] + sx[...]
    om2[...] = adT @ sa[...]
    rhs = (rh0, rh1, rh2, rh3)
    chs = (ch0, ch1, ch2, ch3)
    acc = jnp.zeros((256, 16), F32)
    for hh in range(4):
        wh = gw[pl.ds(hh * 64, 64), :]
        acc = acc + chs[hh][...] @ (wh @ rhs[hh][...])
    owbdT[...] = acc


def _k1_body(xr, m1, m2, oasx, oadp):
    xb = xr[...]
    oasx[...] = xb @ m1[...]
    oadp[...] = xb @ m2[...]


def _k2_body(g0, g1, asxr, adpr, wbdT, w1a, w1b, bias, pd, pr, ps, eh, ef, pi,
             p4, p48, ohs, odinv):
    m = g0[0] + g1[0]
    av = asxr[...]
    dv = adpr[...]
    vad = (av + dv) @ p4[...]
    exb = jnp.exp(jnp.maximum(vad, 0.2 * vad))
    xb = av @ p48[...]
    den256 = m @ pd[...] + exb @ pr[...]
    s16 = m @ ps[...] + (exb @ eh[...]) * (xb @ ef[...])
    hg = _dotT(s16, wbdT[...]) / (den256 + 1e-16) + bias[...]
    hg = _gelu(hg)
    dinv = lax.rsqrt(m @ pi[...] + 1.0)
    d1 = dinv[:, 0:1]
    ohs[0] = (hg @ w1a[...]) * d1
    ohs[1] = (hg @ w1b[...]) * d1
    odinv[...] = dinv


def _k3_body(a0, a1, hsp, dinv8, b1, w2a, w2b, e0, e1, ohs):
    d1 = dinv8[:, 0:1]
    u0 = a0[0] + hsp[0]
    u1 = a1[0] + hsp[1]
    out1 = d1 * (u0 @ e0[...] + u1 @ e1[...]) + b1[...]
    h2 = _gelu(out1)
    ohs[0] = (h2 @ w2a[...]) * d1
    ohs[1] = (h2 @ w2b[...]) * d1


def _k4_body(a0, a1, hsn, dinv8, b2, e0, e1, out):
    d1 = dinv8[:, 0:1]
    u0 = a0[0] + hsn[0]
    u1 = a1[0] + hsn[1]
    out[...] = d1 * (u0 @ e0[...] + u1 @ e1[...]) + b2[...]


def kernel(x, edge_index, gat_W, gat_att_src, gat_att_dst, gat_bias,
           gcn1_W, gcn1_b, gcn2_W, gcn2_b):
    src = edge_index[0]
    dst = edge_index[1]
    npad = E_PAD - E
    src_p = jnp.concatenate([src, jnp.full((npad,), N, jnp.int32)])
    dst_p = jnp.concatenate([dst, jnp.full((npad,), N, jnp.int32)])
    srcs2 = jnp.stack([src_p, src_p + TR])

    m1, m2, wbdT = pl.pallas_call(
        _k0_body,
        grid=(1,),
        in_specs=[_full((H * C, F)), _full((H * C, 1)), _full((H * C, 1)),
                  _full((256, 4)), _full((4, 16)), _full((4, 16)),
                  *[_full((4, 16))] * 4, *[_full((256, 64))] * 4],
        out_specs=[_full((4, 16)), _full((4, 16)), _full((256, 16))],
        out_shape=[jax.ShapeDtypeStruct((4, 16), F32),
                   jax.ShapeDtypeStruct((4, 16), F32),
                   jax.ShapeDtypeStruct((256, 16), F32)],
    )(gat_W, gat_att_src.reshape(H * C, 1), gat_att_dst.reshape(H * C, 1),
      _PRT, _SA, _SX, *_RHT, *_CHT)

    asx, adp = pl.pallas_call(
        _k1_body,
        grid=(GRID,),
        in_specs=[_rows(4), _full((4, 16)), _full((4, 16))],
        out_specs=[_rows(16), _rows(16)],
        out_shape=[jax.ShapeDtypeStruct((TR, L), F32),
                   jax.ShapeDtypeStruct((TR, L), F32)],
    )(x, m1, m2)

    gat_p = _gat_pass(asx, adp, src_p, dst_p)

    hs2, dinv8 = pl.pallas_call(
        _k2_body,
        grid=(GRID,),
        in_specs=[_core(0, GW), _core(1, GW), _rows(16), _rows(16),
                  _full((256, 16)), _full((256, 32)), _full((256, 32)),
                  _full((1, 256)), _full((GW, 256)), _full((4, 256)),
                  _full((GW, 16)), _full((4, 16)), _full((4, 16)),
                  _full((GW, 8)), _full((16, 4)), _full((16, 4))],
        out_specs=[pl.BlockSpec((2, BK, 32), lambda i: (0, i, 0)), _rows(8)],
        out_shape=[jax.ShapeDtypeStruct((2, TR, 32), F32),
                   jax.ShapeDtypeStruct((TR, 8), F32)],
    )(gat_p, gat_p, asx, adp, wbdT, gcn1_W.T[:, 0:32], gcn1_W.T[:, 32:64],
      gat_bias.reshape(1, H * C), _PD, _PR, _PS, _EH, _EF, _PI, _P4, _P48)

    acc1 = _gcn_pass(hs2.reshape(2 * TR, 32), srcs2, dst_p)

    hs2b = pl.pallas_call(
        _k3_body,
        grid=(GRID,),
        in_specs=[_core(0), _core(1),
                  pl.BlockSpec((2, BK, 32), lambda i: (0, i, 0)), _rows(8),
                  _full((1, 64)), _full((64, 32)), _full((64, 32)),
                  _full((32, 64)), _full((32, 64))],
        out_specs=pl.BlockSpec((2, BK, 32), lambda i: (0, i, 0)),
        out_shape=jax.ShapeDtypeStruct((2, TR, 32), F32),
    )(acc1, acc1, hs2, dinv8, gcn1_b.reshape(1, C),
      gcn2_W.T[:, 0:32], gcn2_W.T[:, 32:64], _E0, _E1)

    acc2 = _gcn_pass(hs2b.reshape(2 * TR, 32), srcs2, dst_p)

    out = pl.pallas_call(
        _k4_body,
        grid=(GRID,),
        in_specs=[_core(0), _core(1),
                  pl.BlockSpec((2, BK, 32), lambda i: (0, i, 0)), _rows(8),
                  _full((1, 64)), _full((32, 64)), _full((32, 64))],
        out_specs=_rows(64),
        out_shape=jax.ShapeDtypeStruct((N, C), F32),
    )(acc2, acc2, hs2b, dinv8, gcn2_b.reshape(1, C), _E0, _E1)
    return out


# speedup vs baseline: 1.0022x; 1.0022x over previous
import functools

import jax
import jax.numpy as jnp
from jax import lax
from jax.experimental import pallas as pl
from jax.experimental.pallas import tpu as pltpu
from jax.experimental.pallas import tpu_sc as plsc

N = 50000
E = 800000
F = 4
C = 64
H = 4

NC, NS, L = 2, 16, 16
NW = NC * NS
CH = 256
GW = 24
CHG = 256
EPT = 25088
E_PAD = EPT * NW
NCHUNK = EPT // CH
TR = 50176
NP = 51200
RPT = NP // NS
ZB = 128
F32 = jnp.float32

_mesh = plsc.VectorSubcoreMesh(core_axis_name="c", subcore_axis_name="s")
_params = pltpu.CompilerParams(use_tc_tiling_on_sc=False)


def _zero_acc(zb, acc, s, width):
    z = jnp.zeros((L,), F32)
    offs = sorted({0, width - L})

    def zrow(i, _):
        for j in offs:
            zb[i, pl.ds(j, L)] = z
        return 0
    lax.fori_loop(0, ZB, zrow, 0)

    def zcp(j, _):
        pltpu.sync_copy(zb, acc.at[pl.ds(s * RPT + j * ZB, ZB)])
        return 0
    lax.fori_loop(0, RPT // ZB, zcp, 0)


def _dump_acc(zb, acc, out, c, s):
    def dcp(j, _):
        r = s * RPT + j * ZB
        pltpu.sync_copy(acc.at[pl.ds(r, ZB)], zb)
        pltpu.sync_copy(zb, out.at[c, pl.ds(r, ZB)])
        return 0
    lax.fori_loop(0, RPT // ZB, dcp, 0)


@functools.partial(
    pl.kernel,
    out_type=jax.ShapeDtypeStruct((NC, NP, GW), F32),
    mesh=_mesh,
    compiler_params=_params,
    scratch_types=[
        pltpu.VMEM((4, CH), jnp.int32),
        pltpu.VMEM((4, CH), jnp.int32),
        pltpu.VMEM((2, CH, L), F32),
        pltpu.VMEM((2, CH, L), F32),
        pltpu.VMEM((2, CH, GW), F32),
        pltpu.VMEM((ZB, GW), F32),
        pltpu.VMEM_SHARED((NP, GW), F32),
        pltpu.SemaphoreType.DMA,
        pltpu.SemaphoreType.DMA,
        pltpu.SemaphoreType.DMA,
    ],
)
def _gat_pass(asx, ad, srcs, dsts, out,
              isx, isd, rows_s, rows_d, vals, zb, acc, isem, gsem, ssem):
    c = lax.axis_index("c")
    s = lax.axis_index("s")
    w = c * NS + s
    base = w * EPT
    _zero_acc(zb, acc, s, GW)
    plsc.subcore_barrier()

    lane = lax.iota(jnp.int32, L)
    m4 = lane < 4
    cnt = jnp.where(lane == 4, 1.0, 0.0).astype(F32)
    i1 = lax.shift_right_logical(lane, 2)
    i2 = 4 + (lane & 3)

    def idx_start(j):
        r = j & 3
        eb = base + j * CH
        pltpu.async_copy(srcs.at[pl.ds(eb, CH)], isx.at[r], isem)
        pltpu.async_copy(dsts.at[pl.ds(eb, CH)], isd.at[r], isem)

    def idx_drain(r):
        pltpu.make_async_copy(srcs.at[pl.ds(0, CH)], isx.at[r], isem).wait()
        pltpu.make_async_copy(dsts.at[pl.ds(0, CH)], isd.at[r], isem).wait()

    def gather_start(p, r):
        pltpu.async_copy(asx.at[isx.at[r]], rows_s.at[p], gsem)
        pltpu.async_copy(ad.at[isd.at[r]], rows_d.at[p], gsem)

    idx_start(0)
    idx_drain(0)
    gather_start(0, 0)
    idx_start(1)

    def body(i, _):
        p = i & 1
        q = 1 - p
        pltpu.make_async_copy(asx.at[pl.ds(0, CH)], rows_s.at[p], gsem).wait()
        pltpu.make_async_copy(ad.at[pl.ds(0, CH)], rows_d.at[p], gsem).wait()
        @pl.when(i >= 2)
        def _():
            pltpu.make_async_copy(out.at[0, pl.ds(0, CH)], vals.at[p], ssem).wait()
        @pl.when(i + 2 < NCHUNK)
        def _():
            idx_start(i + 2)
        @pl.when(i + 1 < NCHUNK)
        def _():
            idx_drain((i + 1) & 3)
            gather_start(q, (i + 1) & 3)

        def edge(e, _):
            af = rows_s[p, e, :] + rows_d[p, e, :]
            ex = jnp.exp(jnp.maximum(af, 0.2 * af))
            vals[p, e, pl.ds(0, L)] = jnp.where(m4, ex, cnt)
            g1 = ex.at[i1].get(mode="promise_in_bounds")
            g2 = af.at[i2].get(mode="promise_in_bounds")
            vals[p, e, pl.ds(GW - L, L)] = g1 * g2
            return 0
        lax.fori_loop(0, CH, edge, 0, unroll=16)
        pltpu.async_copy(vals.at[p], acc.at[isd.at[i & 3]], ssem, add=True)
        return 0
    lax.fori_loop(0, NCHUNK, body, 0)
    pltpu.make_async_copy(out.at[0, pl.ds(0, CH)], vals.at[0], ssem).wait()
    pltpu.make_async_copy(out.at[0, pl.ds(0, CH)], vals.at[1], ssem).wait()
    plsc.subcore_barrier()
    _dump_acc(zb, acc, out, c, s)


NCH_G = (E_PAD // NS) // CHG


@functools.partial(
    pl.kernel,
    out_type=jax.ShapeDtypeStruct((NC, NP, 32), F32),
    mesh=_mesh,
    compiler_params=_params,
    scratch_types=[
        pltpu.VMEM((4, CHG), jnp.int32),
        pltpu.VMEM((4, CHG), jnp.int32),
        pltpu.VMEM((2, CHG, 32), F32),
        pltpu.VMEM((ZB, 32), F32),
        pltpu.VMEM_SHARED((NP, 32), F32),
        pltpu.SemaphoreType.DMA,
        pltpu.SemaphoreType.DMA,
        pltpu.SemaphoreType.DMA,
    ],
)
def _gcn_pass(hs2, srcs2, dsts, out, isx, isd, rows, zb, acc, isem, gsem, ssem):
    c = lax.axis_index("c")
    s = lax.axis_index("s")
    base = s * (E_PAD // NS)
    _zero_acc(zb, acc, s, 32)
    plsc.subcore_barrier()

    def idx_start(j):
        r = j & 3
        eb = base + j * CHG
        pltpu.async_copy(srcs2.at[c, pl.ds(eb, CHG)], isx.at[r], isem)
        pltpu.async_copy(dsts.at[pl.ds(eb, CHG)], isd.at[r], isem)

    def idx_drain(r):
        pltpu.make_async_copy(dsts.at[pl.ds(0, CHG)], isx.at[r], isem).wait()
        pltpu.make_async_copy(dsts.at[pl.ds(0, CHG)], isd.at[r], isem).wait()

    idx_start(0)
    idx_drain(0)
    pltpu.async_copy(hs2.at[isx.at[0]], rows.at[0], gsem)
    idx_start(1)

    def body(i, _):
        p = i & 1
        pltpu.make_async_copy(hs2.at[pl.ds(0, CHG)], rows.at[p], gsem).wait()
        @pl.when(i >= 1)
        def _():
            pltpu.make_async_copy(out.at[0, pl.ds(0, CHG)], rows.at[1 - p], ssem).wait()
        @pl.when(i + 2 < NCH_G)
        def _():
            idx_start(i + 2)
        @pl.when(i + 1 < NCH_G)
        def _():
            idx_drain((i + 1) & 3)
            pltpu.async_copy(hs2.at[isx.at[(i + 1) & 3]], rows.at[1 - p], gsem)
        pltpu.async_copy(rows.at[p], acc.at[isd.at[i & 3]], ssem, add=True)
        return 0
    lax.fori_loop(0, NCH_G, body, 0)
    pltpu.make_async_copy(out.at[0, pl.ds(0, CHG)], rows.at[0], ssem).wait()
    plsc.subcore_barrier()
    _dump_acc(zb, acc, out, c, s)

BK = 3584
GRID = TR // BK


def _sel(shape, fill):
    import numpy as np
    m = np.zeros(shape, np.float32)
    fill(m)
    return m


_PD = _sel((GW, 256), lambda m: [m.__setitem__((hh, slice(hh * 64, hh * 64 + 64)), 1.0) for hh in range(4)])
_PR = _sel((4, 256), lambda m: [m.__setitem__((hh, slice(hh * 64, hh * 64 + 64)), 1.0) for hh in range(4)])
_PS = _sel((GW, 16), lambda m: [m.__setitem__((GW - 16 + k, k), 1.0) for k in range(16)])
_EH = _sel((4, 16), lambda m: [m.__setitem__((hh, hh * 4 + ff), 1.0) for hh in range(4) for ff in range(4)])
_EF = _sel((4, 16), lambda m: [m.__setitem__((ff, hh * 4 + ff), 1.0) for hh in range(4) for ff in range(4)])
_PI = _sel((GW, 8), lambda m: m.__setitem__((4, slice(None)), 1.0))
_E0 = _sel((32, 64), lambda m: [m.__setitem__((j, j), 1.0) for j in range(32)])
_E1 = _sel((32, 64), lambda m: [m.__setitem__((j, 32 + j), 1.0) for j in range(32)])
_P4 = _sel((16, 4), lambda m: [m.__setitem__((k, k), 1.0) for k in range(4)])
_P48 = _sel((16, 4), lambda m: [m.__setitem__((4 + k, k), 1.0) for k in range(4)])
_SA = _sel((4, 16), lambda m: [m.__setitem__((k, k), 1.0) for k in range(4)])
_SX = _sel((4, 16), lambda m: [m.__setitem__((k, 4 + k), 1.0) for k in range(4)])
_PRT = _PR.T.copy()
_RHT = [_sel((4, 16), lambda m, h=h: [m.__setitem__((ff, h * 4 + ff), 1.0) for ff in range(4)]) for h in range(4)]
_CHT = [_sel((256, 64), lambda m, h=h: [m.__setitem__((h * 64 + cc, cc), 1.0) for cc in range(64)]) for h in range(4)]


def _full(shape):
    return pl.BlockSpec(shape, lambda i: tuple(0 for _ in shape))


def _rows(width):
    return pl.BlockSpec((BK, width), lambda i: (i, 0))


def _core(cc, width=32):
    return pl.BlockSpec((1, BK, width), lambda i, c=cc: (c, i, 0))


def _gelu(v):
    return 0.5 * v * (1.0 + lax.erf(v * (2.0 ** -0.5)))


def _dotT(a, bT):
    return lax.dot_general(a, bT, (((1,), (1,)), ((), ())))


def _k0_body(gw, afs, afd, prt, sa, sx, rh0, rh1, rh2, rh3,
             ch0, ch1, ch2, ch3, om1, om2, owbdT):
    gwv = gw[...]
    blk_s = afs[...] * prt[...]
    blk_d = afd[...] * prt[...]
    asT = lax.dot_general(gwv, blk_s, (((0,), (0,)), ((), ())))
    adT = lax.dot_general(gwv, blk_d, (((0,), (0,)), ((), ())))
    om1[...] = asT @ sa[...] + sx[...]
    om2[...] = adT @ sa[...]
    rhs = (rh0, rh1, rh2, rh3)
    chs = (ch0, ch1, ch2, ch3)
    acc = jnp.zeros((256, 16), F32)
    for hh in range(4):
        wh = gw[pl.ds(hh * 64, 64), :]
        acc = acc + chs[hh][...] @ (wh @ rhs[hh][...])
    owbdT[...] = acc


def _k1_body(xr, m1, m2, oasx, oadp):
    xb = xr[...]
    oasx[...] = xb @ m1[...]
    oadp[...] = xb @ m2[...]


def _k2_body(g0, g1, asxr, adpr, wbdT, w1a, w1b, bias, pd, pr, ps, eh, ef, pi,
             p4, p48, ohs, odinv):
    m = g0[0] + g1[0]
    av = asxr[...]
    dv = adpr[...]
    vad = (av + dv) @ p4[...]
    exb = jnp.exp(jnp.maximum(vad, 0.2 * vad))
    xb = av @ p48[...]
    den256 = m @ pd[...] + exb @ pr[...]
    s16 = m @ ps[...] + (exb @ eh[...]) * (xb @ ef[...])
    hg = _dotT(s16, wbdT[...]) / (den256 + 1e-16) + bias[...]
    hg = _gelu(hg)
    dinv = lax.rsqrt(m @ pi[...] + 1.0)
    d1 = dinv[:, 0:1]
    ohs[0] = (hg @ w1a[...]) * d1
    ohs[1] = (hg @ w1b[...]) * d1
    odinv[...] = dinv


def _k3_body(a0, a1, hsp, dinv8, b1, w2a, w2b, e0, e1, ohs):
    d1 = dinv8[:, 0:1]
    u0 = a0[0] + hsp[0]
    u1 = a1[0] + hsp[1]
    out1 = d1 * (u0 @ e0[...] + u1 @ e1[...]) + b1[...]
    h2 = _gelu(out1)
    ohs[0] = (h2 @ w2a[...]) * d1
    ohs[1] = (h2 @ w2b[...]) * d1


def _k4_body(a0, a1, hsn, dinv8, b2, e0, e1, out):
    d1 = dinv8[:, 0:1]
    u0 = a0[0] + hsn[0]
    u1 = a1[0] + hsn[1]
    out[...] = d1 * (u0 @ e0[...] + u1 @ e1[...]) + b2[...]


def kernel(x, edge_index, gat_W, gat_att_src, gat_att_dst, gat_bias,
           gcn1_W, gcn1_b, gcn2_W, gcn2_b):
    src = edge_index[0]
    dst = edge_index[1]
    npad = E_PAD - E
    src_p = jnp.concatenate([src, jnp.full((npad,), N, jnp.int32)])
    dst_p = jnp.concatenate([dst, jnp.full((npad,), N, jnp.int32)])
    srcs2 = jnp.stack([src_p, src_p + TR])

    m1, m2, wbdT = pl.pallas_call(
        _k0_body,
        grid=(1,),
        in_specs=[_full((H * C, F)), _full((H * C, 1)), _full((H * C, 1)),
                  _full((256, 4)), _full((4, 16)), _full((4, 16)),
                  *[_full((4, 16))] * 4, *[_full((256, 64))] * 4],
        out_specs=[_full((4, 16)), _full((4, 16)), _full((256, 16))],
        out_shape=[jax.ShapeDtypeStruct((4, 16), F32),
                   jax.ShapeDtypeStruct((4, 16), F32),
                   jax.ShapeDtypeStruct((256, 16), F32)],
    )(gat_W, gat_att_src.reshape(H * C, 1), gat_att_dst.reshape(H * C, 1),
      _PRT, _SA, _SX, *_RHT, *_CHT)

    asx, adp = pl.pallas_call(
        _k1_body,
        grid=(GRID,),
        in_specs=[_rows(4), _full((4, 16)), _full((4, 16))],
        out_specs=[_rows(16), _rows(16)],
        out_shape=[jax.ShapeDtypeStruct((TR, L), F32),
                   jax.ShapeDtypeStruct((TR, L), F32)],
    )(x, m1, m2)

    gat_p = _gat_pass(asx, adp, src_p, dst_p)

    hs2, dinv8 = pl.pallas_call(
        _k2_body,
        grid=(GRID,),
        in_specs=[_core(0, GW), _core(1, GW), _rows(16), _rows(16),
                  _full((256, 16)), _full((256, 32)), _full((256, 32)),
                  _full((1, 256)), _full((GW, 256)), _full((4, 256)),
                  _full((GW, 16)), _full((4, 16)), _full((4, 16)),
                  _full((GW, 8)), _full((16, 4)), _full((16, 4))],
        out_specs=[pl.BlockSpec((2, BK, 32), lambda i: (0, i, 0)), _rows(8)],
        out_shape=[jax.ShapeDtypeStruct((2, TR, 32), F32),
                   jax.ShapeDtypeStruct((TR, 8), F32)],
    )(gat_p, gat_p, asx, adp, wbdT, gcn1_W.T[:, 0:32], gcn1_W.T[:, 32:64],
      gat_bias.reshape(1, H * C), _PD, _PR, _PS, _EH, _EF, _PI, _P4, _P48)

    acc1 = _gcn_pass(hs2.reshape(2 * TR, 32), srcs2, dst_p)

    hs2b = pl.pallas_call(
        _k3_body,
        grid=(GRID,),
        in_specs=[_core(0), _core(1),
                  pl.BlockSpec((2, BK, 32), lambda i: (0, i, 0)), _rows(8),
                  _full((1, 64)), _full((64, 32)), _full((64, 32)),
                  _full((32, 64)), _full((32, 64))],
        out_specs=pl.BlockSpec((2, BK, 32), lambda i: (0, i, 0)),
        out_shape=jax.ShapeDtypeStruct((2, TR, 32), F32),
    )(acc1, acc1, hs2, dinv8, gcn1_b.reshape(1, C),
      gcn2_W.T[:, 0:32], gcn2_W.T[:, 32:64], _E0, _E1)

    acc2 = _gcn_pass(hs2b.reshape(2 * TR, 32), srcs2, dst_p)

    out = pl.pallas_call(
        _k4_body,
        grid=(GRID,),
        in_specs=[_core(0), _core(1),
                  pl.BlockSpec((2, BK, 32), lambda i: (0, i, 0)), _rows(8),
                  _full((1, 64)), _full((32, 64)), _full((32, 64))],
        out_specs=_rows(64),
        out_shape=jax.ShapeDtypeStruct((N, C), F32),
    )(acc2, acc2, hs2b, dinv8, gcn2_b.reshape(1, C), _E0, _E1)
    return out


# speedup vs baseline: 1.0027x; 1.0005x over previous
import functools

import jax
import jax.numpy as jnp
from jax import lax
from jax.experimental import pallas as pl
from jax.experimental.pallas import tpu as pltpu
from jax.experimental.pallas import tpu_sc as plsc

N = 50000
E = 800000
F = 4
C = 64
H = 4

NC, NS, L = 2, 16, 16
NW = NC * NS
CH = 256
GW = 24
CHG = 256
EPT = 25088
E_PAD = EPT * NW
NCHUNK = EPT // CH
TR = 50176
NP = 51200
RPT = NP // NS
ZB = 128
F32 = jnp.float32

_mesh = plsc.VectorSubcoreMesh(core_axis_name="c", subcore_axis_name="s",
                               num_cores=NC, num_subcores=NS)
_params = pltpu.CompilerParams(use_tc_tiling_on_sc=False)


def _zero_acc(zb, acc, s, width):
    z = jnp.zeros((L,), F32)
    offs = sorted({0, width - L})

    def zrow(i, _):
        for j in offs:
            zb[i, pl.ds(j, L)] = z
        return 0
    lax.fori_loop(0, ZB, zrow, 0)

    def zcp(j, _):
        pltpu.sync_copy(zb, acc.at[pl.ds(s * RPT + j * ZB, ZB)])
        return 0
    lax.fori_loop(0, RPT // ZB, zcp, 0)


def _dump_acc(zb, acc, out, c, s):
    def dcp(j, _):
        r = s * RPT + j * ZB
        pltpu.sync_copy(acc.at[pl.ds(r, ZB)], zb)
        pltpu.sync_copy(zb, out.at[c, pl.ds(r, ZB)])
        return 0
    lax.fori_loop(0, RPT // ZB, dcp, 0)


@functools.partial(
    pl.kernel,
    out_type=jax.ShapeDtypeStruct((NC, NP, GW), F32),
    mesh=_mesh,
    compiler_params=_params,
    scratch_types=[
        pltpu.VMEM((4, CH), jnp.int32),
        pltpu.VMEM((4, CH), jnp.int32),
        pltpu.VMEM((2, CH, L), F32),
        pltpu.VMEM((2, CH, L), F32),
        pltpu.VMEM((2, CH, GW), F32),
        pltpu.VMEM((ZB, GW), F32),
        pltpu.VMEM_SHARED((NP, GW), F32),
        pltpu.SemaphoreType.DMA,
        pltpu.SemaphoreType.DMA,
        pltpu.SemaphoreType.DMA,
    ],
)
def _gat_pass(asx, ad, srcs, dsts, out,
              isx, isd, rows_s, rows_d, vals, zb, acc, isem, gsem, ssem):
    c = lax.axis_index("c")
    s = lax.axis_index("s")
    w = c * NS + s
    base = w * EPT
    _zero_acc(zb, acc, s, GW)
    plsc.subcore_barrier()

    lane = lax.iota(jnp.int32, L)
    m4 = lane < 4
    cnt = jnp.where(lane == 4, 1.0, 0.0).astype(F32)
    i1 = lax.shift_right_logical(lane, 2)
    i2 = 4 + (lane & 3)

    def idx_start(j):
        r = j & 3
        eb = base + j * CH
        pltpu.async_copy(srcs.at[pl.ds(eb, CH)], isx.at[r], isem)
        pltpu.async_copy(dsts.at[pl.ds(eb, CH)], isd.at[r], isem)

    def idx_drain(r):
        pltpu.make_async_copy(srcs.at[pl.ds(0, CH)], isx.at[r], isem).wait()
        pltpu.make_async_copy(dsts.at[pl.ds(0, CH)], isd.at[r], isem).wait()

    def gather_start(p, r):
        pltpu.async_copy(asx.at[isx.at[r]], rows_s.at[p], gsem)
        pltpu.async_copy(ad.at[isd.at[r]], rows_d.at[p], gsem)

    idx_start(0)
    idx_drain(0)
    gather_start(0, 0)
    idx_start(1)

    def body(i, _):
        p = i & 1
        q = 1 - p
        pltpu.make_async_copy(asx.at[pl.ds(0, CH)], rows_s.at[p], gsem).wait()
        pltpu.make_async_copy(ad.at[pl.ds(0, CH)], rows_d.at[p], gsem).wait()
        @pl.when(i >= 2)
        def _():
            pltpu.make_async_copy(out.at[0, pl.ds(0, CH)], vals.at[p], ssem).wait()
        @pl.when(i + 2 < NCHUNK)
        def _():
            idx_start(i + 2)
        @pl.when(i + 1 < NCHUNK)
        def _():
            idx_drain((i + 1) & 3)
            gather_start(q, (i + 1) & 3)

        def edge(e, _):
            af = rows_s[p, e, :] + rows_d[p, e, :]
            ex = jnp.exp(jnp.maximum(af, 0.2 * af))
            vals[p, e, pl.ds(0, L)] = jnp.where(m4, ex, cnt)
            g1 = ex.at[i1].get(mode="promise_in_bounds")
            g2 = af.at[i2].get(mode="promise_in_bounds")
            vals[p, e, pl.ds(GW - L, L)] = g1 * g2
            return 0
        lax.fori_loop(0, CH, edge, 0, unroll=16)
        pltpu.async_copy(vals.at[p], acc.at[isd.at[i & 3]], ssem, add=True)
        return 0
    lax.fori_loop(0, NCHUNK, body, 0)
    pltpu.make_async_copy(out.at[0, pl.ds(0, CH)], vals.at[0], ssem).wait()
    pltpu.make_async_copy(out.at[0, pl.ds(0, CH)], vals.at[1], ssem).wait()
    plsc.subcore_barrier()
    _dump_acc(zb, acc, out, c, s)


NCH_G = (E_PAD // NS) // CHG


@functools.partial(
    pl.kernel,
    out_type=jax.ShapeDtypeStruct((NC, NP, 32), F32),
    mesh=_mesh,
    compiler_params=_params,
    scratch_types=[
        pltpu.VMEM((4, CHG), jnp.int32),
        pltpu.VMEM((4, CHG), jnp.int32),
        pltpu.VMEM((2, CHG, 32), F32),
        pltpu.VMEM((ZB, 32), F32),
        pltpu.VMEM_SHARED((NP, 32), F32),
        pltpu.SemaphoreType.DMA,
        pltpu.SemaphoreType.DMA,
        pltpu.SemaphoreType.DMA,
    ],
)
def _gcn_pass(hs2, srcs2, dsts, out, isx, isd, rows, zb, acc, isem, gsem, ssem):
    c = lax.axis_index("c")
    s = lax.axis_index("s")
    base = s * (E_PAD // NS)
    _zero_acc(zb, acc, s, 32)
    plsc.subcore_barrier()

    def idx_start(j):
        r = j & 3
        eb = base + j * CHG
        pltpu.async_copy(srcs2.at[c, pl.ds(eb, CHG)], isx.at[r], isem)
        pltpu.async_copy(dsts.at[pl.ds(eb, CHG)], isd.at[r], isem)

    def idx_drain(r):
        pltpu.make_async_copy(dsts.at[pl.ds(0, CHG)], isx.at[r], isem).wait()
        pltpu.make_async_copy(dsts.at[pl.ds(0, CHG)], isd.at[r], isem).wait()

    idx_start(0)
    idx_drain(0)
    pltpu.async_copy(hs2.at[isx.at[0]], rows.at[0], gsem)
    idx_start(1)

    def body(i, _):
        p = i & 1
        pltpu.make_async_copy(hs2.at[pl.ds(0, CHG)], rows.at[p], gsem).wait()
        @pl.when(i >= 1)
        def _():
            pltpu.make_async_copy(out.at[0, pl.ds(0, CHG)], rows.at[1 - p], ssem).wait()
        @pl.when(i + 2 < NCH_G)
        def _():
            idx_start(i + 2)
        @pl.when(i + 1 < NCH_G)
        def _():
            idx_drain((i + 1) & 3)
            pltpu.async_copy(hs2.at[isx.at[(i + 1) & 3]], rows.at[1 - p], gsem)
        pltpu.async_copy(rows.at[p], acc.at[isd.at[i & 3]], ssem, add=True)
        return 0
    lax.fori_loop(0, NCH_G, body, 0)
    pltpu.make_async_copy(out.at[0, pl.ds(0, CHG)], rows.at[0], ssem).wait()
    plsc.subcore_barrier()
    _dump_acc(zb, acc, out, c, s)

BK = 3584
GRID = TR // BK


def _sel(shape, fill):
    import numpy as np
    m = np.zeros(shape, np.float32)
    fill(m)
    return m


_PD = _sel((GW, 256), lambda m: [m.__setitem__((hh, slice(hh * 64, hh * 64 + 64)), 1.0) for hh in range(4)])
_PR = _sel((4, 256), lambda m: [m.__setitem__((hh, slice(hh * 64, hh * 64 + 64)), 1.0) for hh in range(4)])
_PS = _sel((GW, 16), lambda m: [m.__setitem__((GW - 16 + k, k), 1.0) for k in range(16)])
_EH = _sel((4, 16), lambda m: [m.__setitem__((hh, hh * 4 + ff), 1.0) for hh in range(4) for ff in range(4)])
_EF = _sel((4, 16), lambda m: [m.__setitem__((ff, hh * 4 + ff), 1.0) for hh in range(4) for ff in range(4)])
_PI = _sel((GW, 8), lambda m: m.__setitem__((4, slice(None)), 1.0))
_E0 = _sel((32, 64), lambda m: [m.__setitem__((j, j), 1.0) for j in range(32)])
_E1 = _sel((32, 64), lambda m: [m.__setitem__((j, 32 + j), 1.0) for j in range(32)])
_P4 = _sel((16, 4), lambda m: [m.__setitem__((k, k), 1.0) for k in range(4)])
_P48 = _sel((16, 4), lambda m: [m.__setitem__((4 + k, k), 1.0) for k in range(4)])
_SA = _sel((4, 16), lambda m: [m.__setitem__((k, k), 1.0) for k in range(4)])
_SX = _sel((4, 16), lambda m: [m.__setitem__((k, 4 + k), 1.0) for k in range(4)])
_PRT = _PR.T.copy()
_RHT = [_sel((4, 16), lambda m, h=h: [m.__setitem__((ff, h * 4 + ff), 1.0) for ff in range(4)]) for h in range(4)]
_CHT = [_sel((256, 64), lambda m, h=h: [m.__setitem__((h * 64 + cc, cc), 1.0) for cc in range(64)]) for h in range(4)]


def _full(shape):
    return pl.BlockSpec(shape, lambda i: tuple(0 for _ in shape))


def _rows(width):
    return pl.BlockSpec((BK, width), lambda i: (i, 0))


def _core(cc, width=32):
    return pl.BlockSpec((1, BK, width), lambda i, c=cc: (c, i, 0))


def _gelu(v):
    return 0.5 * v * (1.0 + lax.erf(v * (2.0 ** -0.5)))


def _dotT(a, bT):
    return lax.dot_general(a, bT, (((1,), (1,)), ((), ())))


def _k0_body(gw, afs, afd, prt, sa, sx, rh0, rh1, rh2, rh3,
             ch0, ch1, ch2, ch3, om1, om2, owbdT):
    gwv = gw[...]
    blk_s = afs[...] * prt[...]
    blk_d = afd[...] * prt[...]
    asT = lax.dot_general(gwv, blk_s, (((0,), (0,)), ((), ())))
    adT = lax.dot_general(gwv, blk_d, (((0,), (0,)), ((), ())))
    om1[...] = asT @ sa[...] + sx[...]
    om2[...] = adT @ sa[...]
    rhs = (rh0, rh1, rh2, rh3)
    chs = (ch0, ch1, ch2, ch3)
    acc = jnp.zeros((256, 16), F32)
    for hh in range(4):
        wh = gw[pl.ds(hh * 64, 64), :]
        acc = acc + chs[hh][...] @ (wh @ rhs[hh][...])
    owbdT[...] = acc


def _k1_body(xr, m1, m2, oasx, oadp):
    xb = xr[...]
    oasx[...] = xb @ m1[...]
    oadp[...] = xb @ m2[...]


def _k2_body(g0, g1, asxr, adpr, wbdT, w1a, w1b, bias, pd, pr, ps, eh, ef, pi,
             p4, p48, ohs, odinv):
    m = g0[0] + g1[0]
    av = asxr[...]
    dv = adpr[...]
    vad = (av + dv) @ p4[...]
    exb = jnp.exp(jnp.maximum(vad, 0.2 * vad))
    xb = av @ p48[...]
    den256 = m @ pd[...] + exb @ pr[...]
    s16 = m @ ps[...] + (exb @ eh[...]) * (xb @ ef[...])
    hg = _dotT(s16, wbdT[...]) / (den256 + 1e-16) + bias[...]
    hg = _gelu(hg)
    dinv = lax.rsqrt(m @ pi[...] + 1.0)
    d1 = dinv[:, 0:1]
    ohs[0] = (hg @ w1a[...]) * d1
    ohs[1] = (hg @ w1b[...]) * d1
    odinv[...] = dinv


def _k3_body(a0, a1, hsp, dinv8, b1, w2a, w2b, e0, e1, ohs):
    d1 = dinv8[:, 0:1]
    u0 = a0[0] + hsp[0]
    u1 = a1[0] + hsp[1]
    out1 = d1 * (u0 @ e0[...] + u1 @ e1[...]) + b1[...]
    h2 = _gelu(out1)
    ohs[0] = (h2 @ w2a[...]) * d1
    ohs[1] = (h2 @ w2b[...]) * d1


def _k4_body(a0, a1, hsn, dinv8, b2, e0, e1, out):
    d1 = dinv8[:, 0:1]
    u0 = a0[0] + hsn[0]
    u1 = a1[0] + hsn[1]
    out[...] = d1 * (u0 @ e0[...] + u1 @ e1[...]) + b2[...]


def kernel(x, edge_index, gat_W, gat_att_src, gat_att_dst, gat_bias,
           gcn1_W, gcn1_b, gcn2_W, gcn2_b):
    src = edge_index[0]
    dst = edge_index[1]
    npad = E_PAD - E
    src_p = jnp.concatenate([src, jnp.full((npad,), N, jnp.int32)])
    dst_p = jnp.concatenate([dst, jnp.full((npad,), N, jnp.int32)])
    srcs2 = jnp.stack([src_p, src_p + TR])

    m1, m2, wbdT = pl.pallas_call(
        _k0_body,
        grid=(1,),
        in_specs=[_full((H * C, F)), _full((H * C, 1)), _full((H * C, 1)),
                  _full((256, 4)), _full((4, 16)), _full((4, 16)),
                  *[_full((4, 16))] * 4, *[_full((256, 64))] * 4],
        out_specs=[_full((4, 16)), _full((4, 16)), _full((256, 16))],
        out_shape=[jax.ShapeDtypeStruct((4, 16), F32),
                   jax.ShapeDtypeStruct((4, 16), F32),
                   jax.ShapeDtypeStruct((256, 16), F32)],
    )(gat_W, gat_att_src.reshape(H * C, 1), gat_att_dst.reshape(H * C, 1),
      _PRT, _SA, _SX, *_RHT, *_CHT)

    asx, adp = pl.pallas_call(
        _k1_body,
        grid=(GRID,),
        in_specs=[_rows(4), _full((4, 16)), _full((4, 16))],
        out_specs=[_rows(16), _rows(16)],
        out_shape=[jax.ShapeDtypeStruct((TR, L), F32),
                   jax.ShapeDtypeStruct((TR, L), F32)],
    )(x, m1, m2)

    gat_p = _gat_pass(asx, adp, src_p, dst_p)

    hs2, dinv8 = pl.pallas_call(
        _k2_body,
        grid=(GRID,),
        in_specs=[_core(0, GW), _core(1, GW), _rows(16), _rows(16),
                  _full((256, 16)), _full((256, 32)), _full((256, 32)),
                  _full((1, 256)), _full((GW, 256)), _full((4, 256)),
                  _full((GW, 16)), _full((4, 16)), _full((4, 16)),
                  _full((GW, 8)), _full((16, 4)), _full((16, 4))],
        out_specs=[pl.BlockSpec((2, BK, 32), lambda i: (0, i, 0)), _rows(8)],
        out_shape=[jax.ShapeDtypeStruct((2, TR, 32), F32),
                   jax.ShapeDtypeStruct((TR, 8), F32)],
    )(gat_p, gat_p, asx, adp, wbdT, gcn1_W.T[:, 0:32], gcn1_W.T[:, 32:64],
      gat_bias.reshape(1, H * C), _PD, _PR, _PS, _EH, _EF, _PI, _P4, _P48)

    acc1 = _gcn_pass(hs2.reshape(2 * TR, 32), srcs2, dst_p)

    hs2b = pl.pallas_call(
        _k3_body,
        grid=(GRID,),
        in_specs=[_core(0), _core(1),
                  pl.BlockSpec((2, BK, 32), lambda i: (0, i, 0)), _rows(8),
                  _full((1, 64)), _full((64, 32)), _full((64, 32)),
                  _full((32, 64)), _full((32, 64))],
        out_specs=pl.BlockSpec((2, BK, 32), lambda i: (0, i, 0)),
        out_shape=jax.ShapeDtypeStruct((2, TR, 32), F32),
    )(acc1, acc1, hs2, dinv8, gcn1_b.reshape(1, C),
      gcn2_W.T[:, 0:32], gcn2_W.T[:, 32:64], _E0, _E1)

    acc2 = _gcn_pass(hs2b.reshape(2 * TR, 32), srcs2, dst_p)

    out = pl.pallas_call(
        _k4_body,
        grid=(GRID,),
        in_specs=[_core(0), _core(1),
                  pl.BlockSpec((2, BK, 32), lambda i: (0, i, 0)), _rows(8),
                  _full((1, 64)), _full((32, 64)), _full((32, 64))],
        out_specs=_rows(64),
        out_shape=jax.ShapeDtypeStruct((N, C), F32),
    )(acc2, acc2, hs2b, dinv8, gcn2_b.reshape(1, C), _E0, _E1)
    return out
